# Initial kernel scaffold; baseline (speedup 1.0000x reference)
import jax
import jax.numpy as jnp
from jax.experimental import pallas as pl


def kernel(xyz, xyz_id, scales, rotation, knn_index, normal):
    raise NotImplementedError("write your pallas kernel here")


# speedup vs baseline: 6.9970x; 6.9970x over previous
import jax
import jax.numpy as jnp
from jax import lax
from jax.experimental import pallas as pl
from jax.experimental.pallas import tpu as pltpu
from jax.experimental.pallas import tpu_sc as plsc

N = 100000
K = 16
NC = 2
NS = 16
NW = NC * NS
P = 3136
B = 448
NB = P // B
G = B // 16
CH = (B * K) // 128


def _rsqrt(x):
    i = lax.bitcast_convert_type(x, jnp.int32)
    i = jnp.int32(0x5F3759DF) - lax.shift_right_logical(i, 1)
    y = lax.bitcast_convert_type(i, jnp.float32)
    for _ in range(3):
        y = y * (1.5 - 0.5 * x * y * y)
    return y


def _sc_body(xyz4, xyz4f, knnv, scf, rtf, nmf,
             o_size, o_d, o_norm,
             knn_s, rows_s, xyz_s, sc_s, rt_s, nm_s, os_s, od_s, on_s, sem):
    wid = lax.axis_index("s") * NC + lax.axis_index("c")
    iota = lax.iota(jnp.int32, 16)
    cc0 = jnp.zeros((16,), jnp.int32)
    cc1 = jnp.full((16,), 1, jnp.int32)
    cc2 = jnp.full((16,), 2, jnp.int32)

    def block(b, _):
        base = jnp.minimum(wid * P + b * B, N - B)
        pltpu.sync_copy(knnv.at[pl.ds(base * K, B * K)], knn_s)
        pltpu.sync_copy(xyz4f.at[pl.ds(base * 4, B * 4)], xyz_s)
        pltpu.sync_copy(scf.at[pl.ds(base * 3, B * 3)], sc_s)
        pltpu.sync_copy(rtf.at[pl.ds(base * 4, B * 4)], rt_s)
        pltpu.sync_copy(nmf.at[pl.ds(base * 3, B * 3)], nm_s)
        copies = [
            pltpu.async_copy(xyz4.at[knn_s.at[pl.ds(j * 128, 128)]],
                             rows_s.at[pl.ds(j * 128, 128), :], sem)
            for j in range(CH)
        ]
        for c in copies:
            c.wait()

        def group(g, _):
            r3 = (g * 48 + iota * 3)
            r4 = (g * 64 + iota * 4)
            sx = jnp.abs(plsc.load_gather(sc_s, [r3]))
            sy = jnp.abs(plsc.load_gather(sc_s, [r3 + 1]))
            sz = jnp.abs(plsc.load_gather(sc_s, [r3 + 2]))
            qw = plsc.load_gather(rt_s, [r4])
            qx = plsc.load_gather(rt_s, [r4 + 1])
            qy = plsc.load_gather(rt_s, [r4 + 2])
            qz = plsc.load_gather(rt_s, [r4 + 3])
            nx = plsc.load_gather(nm_s, [r3])
            ny = plsc.load_gather(nm_s, [r3 + 1])
            nz = plsc.load_gather(nm_s, [r3 + 2])
            px = plsc.load_gather(xyz_s, [r4])
            py = plsc.load_gather(xyz_s, [r4 + 1])
            pz = plsc.load_gather(xyz_s, [r4 + 2])

            s2 = qw * qw + qx * qx + qy * qy + qz * qz
            sq = s2 * _rsqrt(jnp.maximum(s2, 1e-30))
            inv = 1.0 / (sq + 1e-8)
            w = qw * inv
            x = qx * inv
            y = qy * inv
            z = qz * inv
            xx, yy, zz = x * x, y * y, z * z
            xy, xz, yz = x * y, x * z, y * z
            wx, wy, wz = w * x, w * y, w * z
            c0x, c0y, c0z = 1 - 2 * (yy + zz), 2 * (xy + wz), 2 * (xz - wy)
            c1x, c1y, c1z = 2 * (xy - wz), 1 - 2 * (xx + zz), 2 * (yz + wx)
            c2x, c2y, c2z = 2 * (xz + wy), 2 * (yz - wx), 1 - 2 * (xx + yy)
            is0 = (sx <= sy) & (sx <= sz)
            is1 = jnp.logical_not(is0) & (sy <= sz)
            ngx = jnp.where(is0, c0x, jnp.where(is1, c1x, c2x))
            ngy = jnp.where(is0, c0y, jnp.where(is1, c1y, c2y))
            ngz = jnp.where(is0, c0z, jnp.where(is1, c1z, c2z))

            loss_size = jnp.minimum(sx, jnp.minimum(sy, sz))

            nn = nx * nx + ny * ny + nz * nz
            snn = nn * _rsqrt(jnp.maximum(nn, 1e-30))
            invn = 1.0 / (snn + 1e-8)
            cos = (ngx * nx + ngy * ny + ngz * nz) * invn
            loss_normal = 1.0 - jnp.abs(cos)

            cd = -(px * ngx + py * ngy + pz * ngz)
            rbase = g * 256 + iota * 16
            ds = []
            acc = None
            for k in range(K):
                nbx = plsc.load_gather(rows_s, [rbase + k, cc0])
                nby = plsc.load_gather(rows_s, [rbase + k, cc1])
                nbz = plsc.load_gather(rows_s, [rbase + k, cc2])
                dk = nbx * ngx + nby * ngy + nbz * ngz + cd
                ds.append(dk)
                acc = dk if acc is None else acc + dk
            mean = acc * (1.0 / K)
            aacc = None
            for dk in ds:
                t = jnp.abs(dk - mean)
                aacc = t if aacc is None else aacc + t
            loss_d = aacc * (1.0 / K)

            os_s[pl.ds(g * 16, 16)] = loss_size
            od_s[pl.ds(g * 16, 16)] = loss_d
            on_s[pl.ds(g * 16, 16)] = loss_normal
            return 0

        lax.fori_loop(0, G, group, 0)
        pltpu.sync_copy(os_s, o_size.at[pl.ds(base, B)])
        pltpu.sync_copy(od_s, o_d.at[pl.ds(base, B)])
        pltpu.sync_copy(on_s, o_norm.at[pl.ds(base, B)])
        return 0

    lax.fori_loop(0, NB, block, 0)


@jax.jit
def _run(xyz4, xyz4f, knnv, scf, rtf, nmf):
    f32 = jnp.float32
    out = jax.ShapeDtypeStruct((N,), f32)
    kfn = pl.kernel(
        _sc_body,
        out_type=(out, out, out),
        mesh=plsc.VectorSubcoreMesh(core_axis_name="c", subcore_axis_name="s"),
        compiler_params=pltpu.CompilerParams(needs_layout_passes=False,
                                             use_tc_tiling_on_sc=False),
        scratch_types=[
            pltpu.VMEM((B * K,), jnp.int32),
            pltpu.VMEM((B * K, 4), f32),
            pltpu.VMEM((B * 4,), f32),
            pltpu.VMEM((B * 3,), f32),
            pltpu.VMEM((B * 4,), f32),
            pltpu.VMEM((B * 3,), f32),
            pltpu.VMEM((B,), f32),
            pltpu.VMEM((B,), f32),
            pltpu.VMEM((B,), f32),
            pltpu.SemaphoreType.DMA,
        ],
    )
    return kfn(xyz4, xyz4f, knnv, scf, rtf, nmf)


def kernel(xyz, xyz_id, scales, rotation, knn_index, normal):
    xyz4 = jnp.pad(xyz, ((0, 0), (0, 1)))
    knnv = knn_index.astype(jnp.int32).reshape(N * K)
    return _run(xyz4, xyz4.reshape(N * 4), knnv, scales.reshape(N * 3),
                rotation.reshape(N * 4), normal.reshape(N * 3))
